# Initial kernel scaffold; baseline (speedup 1.0000x reference)
import jax
import jax.numpy as jnp
from jax.experimental import pallas as pl


def kernel(x, edge_index, edge_attr, edge_mark, deg_out, in_W, in_b, msg_W1, msg_b1, msg_W2, msg_b2, up_W1, up_b1, up_W2, up_b2, pred_W1, pred_b1, pred_W2, pred_b2):
    raise NotImplementedError("write your pallas kernel here")


# speedup vs baseline: 1.0823x; 1.0823x over previous
import functools

import jax
import jax.numpy as jnp
from jax import lax
from jax.experimental import pallas as pl
from jax.experimental.pallas import tpu as pltpu
from jax.experimental.pallas import tpu_sc as plsc

N = 10000
E = 320000
H = 128
ED = 16
EM = 2
L = 3

NC = 2
NS = 16
NW = NC * NS
CH = 128

N_PAD = 10240
RPS = N_PAD // NS
CHUNKS = -(-E // (NW * CH))
E_PAD = NW * CH * CHUNKS
EW = CH * CHUNKS

BR = 512
BE = 512


def _in_proj_body(x_ref, w_ref, b_ref, w1h_ref, h_ref, np_ref):
    h = jnp.maximum(jnp.dot(x_ref[...], w_ref[...],
                            preferred_element_type=jnp.float32) + b_ref[...], 0.0)
    h_ref[...] = h
    np_ref[...] = jnp.dot(h, w1h_ref[...], preferred_element_type=jnp.float32)


def _in_proj(x_p, in_W, in_b, w1h0):
    grid = (N_PAD // BR,)
    return pl.pallas_call(
        _in_proj_body,
        grid=grid,
        in_specs=[
            pl.BlockSpec((BR, H), lambda i: (i, 0)),
            pl.BlockSpec((H, H), lambda i: (0, 0)),
            pl.BlockSpec((1, H), lambda i: (0, 0)),
            pl.BlockSpec((H, H), lambda i: (0, 0)),
        ],
        out_specs=[
            pl.BlockSpec((BR, H), lambda i: (i, 0)),
            pl.BlockSpec((BR, H), lambda i: (i, 0)),
        ],
        out_shape=[
            jax.ShapeDtypeStruct((N_PAD, H), jnp.float32),
            jax.ShapeDtypeStruct((N_PAD, H), jnp.float32),
        ],
    )(x_p, in_W, in_b, w1h0)


def _edge_pre_body(ea_ref, em_ref, wa_ref, wm_ref, b1_ref, out_ref):
    acc = jnp.dot(ea_ref[...], wa_ref[0], preferred_element_type=jnp.float32)
    acc = acc + jnp.dot(em_ref[...], wm_ref[0], preferred_element_type=jnp.float32)
    out_ref[0] = acc + b1_ref[0]


def _edge_pre(ea_p, em_p, wa, wm, b1):
    grid = (L, E_PAD // BE)
    return pl.pallas_call(
        _edge_pre_body,
        grid=grid,
        in_specs=[
            pl.BlockSpec((BE, ED), lambda l, e: (e, 0)),
            pl.BlockSpec((BE, EM), lambda l, e: (e, 0)),
            pl.BlockSpec((1, ED, H), lambda l, e: (l, 0, 0)),
            pl.BlockSpec((1, EM, H), lambda l, e: (l, 0, 0)),
            pl.BlockSpec((1, 1, H), lambda l, e: (l, 0, 0)),
        ],
        out_specs=pl.BlockSpec((1, BE, H), lambda l, e: (l, e, 0)),
        out_shape=jax.ShapeDtypeStruct((L, E_PAD, H), jnp.float32),
    )(ea_p, em_p, wa, wm, b1)


def _update_common(pp_ref, deg_ref, h_ref, w2_ref, b2_ref,
                   uw1h_ref, uw1n_ref, ub1_ref, uw2_ref, ub2_ref):
    s = pp_ref[0] + pp_ref[1]
    deg = deg_ref[...]
    deg = jnp.where(deg == 0.0, 1.0, deg)
    neigh = jnp.dot(s / deg, w2_ref[...],
                    preferred_element_type=jnp.float32) + b2_ref[...]
    h = h_ref[...]
    u = jnp.maximum(
        jnp.dot(h, uw1h_ref[...], preferred_element_type=jnp.float32)
        + jnp.dot(neigh, uw1n_ref[...], preferred_element_type=jnp.float32)
        + ub1_ref[...], 0.0)
    u = jnp.dot(u, uw2_ref[...], preferred_element_type=jnp.float32) + ub2_ref[...]
    return jnp.maximum(h + u, 0.0)


def _update_mid_body(pp_ref, deg_ref, h_ref, w2_ref, b2_ref,
                     uw1h_ref, uw1n_ref, ub1_ref, uw2_ref, ub2_ref, w1hn_ref,
                     hn_ref, npn_ref):
    hn = _update_common(pp_ref, deg_ref, h_ref, w2_ref, b2_ref,
                        uw1h_ref, uw1n_ref, ub1_ref, uw2_ref, ub2_ref)
    hn_ref[...] = hn
    npn_ref[...] = jnp.dot(hn, w1hn_ref[...], preferred_element_type=jnp.float32)


def _update_last_body(pp_ref, deg_ref, h_ref, w2_ref, b2_ref,
                      uw1h_ref, uw1n_ref, ub1_ref, uw2_ref, ub2_ref,
                      pw1_ref, pb1_ref, pw2_ref, pb2_ref, p_ref):
    hn = _update_common(pp_ref, deg_ref, h_ref, w2_ref, b2_ref,
                        uw1h_ref, uw1n_ref, ub1_ref, uw2_ref, ub2_ref)
    q = jnp.maximum(jnp.dot(hn, pw1_ref[...],
                            preferred_element_type=jnp.float32) + pb1_ref[...], 0.0)
    p_ref[...] = jnp.dot(q, pw2_ref[...],
                         preferred_element_type=jnp.float32) + pb2_ref[...]


def _wspec(shape):
    nd = len(shape)
    return pl.BlockSpec(shape, lambda i, _nd=nd: (0,) * _nd)


def _update_mid(partial, deg_p, h, w2, b2, uw1h, uw1n, ub1, uw2, ub2, w1hn):
    grid = (N_PAD // BR,)
    return pl.pallas_call(
        _update_mid_body,
        grid=grid,
        in_specs=[
            pl.BlockSpec((NC, BR, H), lambda i: (0, i, 0)),
            pl.BlockSpec((BR, 1), lambda i: (i, 0)),
            pl.BlockSpec((BR, H), lambda i: (i, 0)),
            _wspec((H, H)), _wspec((1, H)),
            _wspec((H, H)), _wspec((H, H)), _wspec((1, H)),
            _wspec((H, H)), _wspec((1, H)), _wspec((H, H)),
        ],
        out_specs=[
            pl.BlockSpec((BR, H), lambda i: (i, 0)),
            pl.BlockSpec((BR, H), lambda i: (i, 0)),
        ],
        out_shape=[
            jax.ShapeDtypeStruct((N_PAD, H), jnp.float32),
            jax.ShapeDtypeStruct((N_PAD, H), jnp.float32),
        ],
    )(partial, deg_p, h, w2, b2, uw1h, uw1n, ub1, uw2, ub2, w1hn)


def _update_last(partial, deg_p, h, w2, b2, uw1h, uw1n, ub1, uw2, ub2,
                 pw1, pb1, pw2, pb2):
    grid = (N_PAD // BR,)
    hh = pw1.shape[1]
    return pl.pallas_call(
        _update_last_body,
        grid=grid,
        in_specs=[
            pl.BlockSpec((NC, BR, H), lambda i: (0, i, 0)),
            pl.BlockSpec((BR, 1), lambda i: (i, 0)),
            pl.BlockSpec((BR, H), lambda i: (i, 0)),
            _wspec((H, H)), _wspec((1, H)),
            _wspec((H, H)), _wspec((H, H)), _wspec((1, H)),
            _wspec((H, H)), _wspec((1, H)),
            _wspec((H, hh)), _wspec((1, hh)), _wspec((hh, 1)), _wspec((1, 1)),
        ],
        out_specs=pl.BlockSpec((BR, 1), lambda i: (i, 0)),
        out_shape=jax.ShapeDtypeStruct((N_PAD, 1), jnp.float32),
    )(partial, deg_p, h, w2, b2, uw1h, uw1n, ub1, uw2, ub2, pw1, pb1, pw2, pb2)


def _sc_scatter_body(np_hbm, ep_hbm, src_hbm, dst_hbm, out_hbm,
                     acc, src_v, dst_v, gath_v, ep_v, sem, ep_base):
    cid = lax.axis_index("c")
    sid = lax.axis_index("s")
    wid = sid * NC + cid

    def zrow(i, _):
        gath_v[i // 8, pl.ds((i % 8) * 16, 16)] = jnp.zeros((16,), jnp.float32)
        return 0
    lax.fori_loop(0, CH * 8, zrow, 0)
    for j in range(RPS // CH):
        pltpu.sync_copy(gath_v, acc.at[pl.ds(sid * RPS + j * CH, CH)])
    plsc.subcore_barrier()

    def chunk(c, _):
        base = wid * EW + c * CH
        pltpu.sync_copy(src_hbm.at[pl.ds(base, CH)], src_v)
        pltpu.sync_copy(dst_hbm.at[pl.ds(base, CH)], dst_v)
        pltpu.async_copy(np_hbm.at[src_v], gath_v, sem).wait()
        pltpu.sync_copy(ep_hbm.at[pl.ds(ep_base + base, CH)], ep_v)

        def crow(i, _):
            r = i // 8
            cc = (i % 8) * 16
            v = gath_v[r, pl.ds(cc, 16)] + ep_v[r, pl.ds(cc, 16)]
            ep_v[r, pl.ds(cc, 16)] = jnp.maximum(v, 0.0)
            return 0
        lax.fori_loop(0, CH * 8, crow, 0)
        pltpu.sync_copy(ep_v, acc.at[dst_v], add=True)
        return 0
    lax.fori_loop(0, CHUNKS, chunk, 0)
    plsc.subcore_barrier()

    pltpu.sync_copy(acc.at[pl.ds(sid * RPS, RPS)],
                    out_hbm.at[cid, pl.ds(sid * RPS, RPS)])


def _sc_scatter(node_pre, ep_flat, src_p, dst_p, layer):
    mesh = plsc.VectorSubcoreMesh(core_axis_name="c", subcore_axis_name="s",
                                  num_cores=NC, num_subcores=NS)
    body = functools.partial(_sc_scatter_body, ep_base=layer * E_PAD)
    return pl.kernel(
        body,
        mesh=mesh,
        out_type=jax.ShapeDtypeStruct((NC, N_PAD, H), jnp.float32),
        scratch_types=[
            pltpu.VMEM_SHARED((N_PAD, H), jnp.float32),
            pltpu.VMEM((CH,), jnp.int32),
            pltpu.VMEM((CH,), jnp.int32),
            pltpu.VMEM((CH, H), jnp.float32),
            pltpu.VMEM((CH, H), jnp.float32),
            pltpu.SemaphoreType.DMA,
        ],
    )(node_pre, ep_flat, src_p, dst_p)


def kernel(x, edge_index, edge_attr, edge_mark, deg_out,
           in_W, in_b,
           msg_W1, msg_b1, msg_W2, msg_b2,
           up_W1, up_b1, up_W2, up_b2,
           pred_W1, pred_b1, pred_W2, pred_b2):
    x_p = jnp.pad(x, ((0, N_PAD - N), (0, 0)))
    deg_p = jnp.pad(deg_out, (0, N_PAD - N)).reshape(N_PAD, 1)
    src_p = jnp.pad(edge_index[0], (0, E_PAD - E))
    dst_p = jnp.pad(edge_index[1], (0, E_PAD - E), constant_values=N)
    ea_p = jnp.pad(edge_attr, ((0, E_PAD - E), (0, 0)))
    em_p = jnp.pad(edge_mark, ((0, E_PAD - E), (0, 0)))

    w1h = msg_W1[:, :H]
    wa = msg_W1[:, H:H + ED]
    wm = msg_W1[:, H + ED:]
    b1 = msg_b1.reshape(L, 1, H)

    h, np_cur = _in_proj(x_p, in_W, in_b.reshape(1, H), w1h[0])
    ep_all = _edge_pre(ea_p, em_p, wa, wm, b1)
    ep_flat = ep_all.reshape(L * E_PAD, H)

    p = None
    for l in range(L):
        partial = _sc_scatter(np_cur, ep_flat, src_p, dst_p, l)
        if l < L - 1:
            h, np_cur = _update_mid(
                partial, deg_p, h, msg_W2[l], msg_b2[l].reshape(1, H),
                up_W1[l, :H], up_W1[l, H:], up_b1[l].reshape(1, H),
                up_W2[l], up_b2[l].reshape(1, H), w1h[l + 1])
        else:
            p = _update_last(
                partial, deg_p, h, msg_W2[l], msg_b2[l].reshape(1, H),
                up_W1[l, :H], up_W1[l, H:], up_b1[l].reshape(1, H),
                up_W2[l], up_b2[l].reshape(1, H),
                pred_W1, pred_b1.reshape(1, -1), pred_W2, pred_b2.reshape(1, 1))
    return p[:N, 0]
